# baseline (speedup 1.0000x reference)
import functools

import jax
import jax.numpy as jnp
from jax import lax
from jax.experimental import pallas as pl
from jax.experimental.pallas import tpu as pltpu
from jax.experimental.pallas import tpu_sc as plsc

T = 4
B = 4096
L = 200
D = 16
N = B * L

NC = 2
NS = 16
NW = NC * NS
CH = 512
NCHUNK_TOTAL = N // CH
PER_W = NCHUNK_TOTAL // NW
NBUF = 2
SCALE = 1.0 / T

_mesh = plsc.VectorSubcoreMesh(
    core_axis_name="c", subcore_axis_name="s", num_cores=NC, num_subcores=NS
)

VOCAB = 1000000
_TC_BC = 16384


def _tc_prep_body(wt_ref, out_ref):
    x = wt_ref[...] * jnp.float32(SCALE)
    xt = jnp.transpose(x)
    x3 = xt.reshape(_TC_BC // 8, 8, D)
    parts = [x3[:, k, :] for k in range(8)]
    out_ref[...] = jnp.concatenate(parts, axis=1)


def _tc_prep(wt):
    nb = pl.cdiv(VOCAB, _TC_BC)
    return pl.pallas_call(
        _tc_prep_body,
        grid=(nb,),
        in_specs=[pl.BlockSpec((D, _TC_BC), lambda i: (0, i))],
        out_specs=pl.BlockSpec((_TC_BC // 8, 128), lambda i: (i, 0)),
        out_shape=jax.ShapeDtypeStruct((VOCAB // 8, 128), jnp.float32),
    )(wt)


NJ = T * B // 128
JCH = CH // 128


TRCH = 2 * JCH * 8 * 128
LSTRIDE = (D // 8) * NJ * 8 * 128
ISTRIDE = NJ * 8 * 128


@functools.partial(
    pl.kernel,
    out_type=jax.ShapeDtypeStruct((L * D * T * B,), jnp.float32),
    mesh=_mesh,
    scratch_types=[
        pltpu.VMEM((NBUF, CH), jnp.int32),
        pltpu.VMEM((NBUF, CH, D), jnp.float32),
        pltpu.VMEM((2 * NBUF, TRCH), jnp.float32),
        pltpu.SemaphoreType.DMA,
        pltpu.SemaphoreType.DMA,
        pltpu.SemaphoreType.DMA,
        pltpu.SemaphoreType.DMA,
        pltpu.SemaphoreType.DMA,
        pltpu.SemaphoreType.DMA,
    ],
    compiler_params=pltpu.CompilerParams(
        use_tc_tiling_on_sc=False, needs_layout_passes=False
    ),
)
def _sc_embed(w_hbm, idx_hbm, out_hbm, idx_v, rows_v, tr_v, g0, g1, s0, s1, s2, s3):
    wid = lax.axis_index("s") * NC + lax.axis_index("c")
    gsem = (g0, g1)
    ssem = (s0, s1, s2, s3)
    lane_iota = lax.iota(jnp.int32, 16)
    base_vec = (
        lax.shift_right_logical(lane_iota, 3) * (JCH * 8 * 128)
        + lax.bitwise_and(lane_iota, 7) * 128
    )

    def fetch(k, buf):
        cg = wid * PER_W + lax.rem(k, PER_W)
        pltpu.sync_copy(idx_hbm.at[pl.ds(cg * CH, CH)], idx_v.at[buf])
        pltpu.async_copy(w_hbm.at[idx_v.at[buf]], rows_v.at[buf], gsem[buf])

    def wait_gather(buf):
        pltpu.make_async_copy(
            w_hbm.at[idx_v.at[buf]], rows_v.at[buf], gsem[buf]
        ).wait()

    def transpose_chunk(buf_r, buf_t):
        def jbody(j, carry):
            def tr_body(c, idx):
                row = rows_v[buf_r, j * 128 + c, :]
                plsc.store_scatter(tr_v.at[buf_t], [idx], row)
                return idx + 1

            lax.fori_loop(0, 128, tr_body, base_vec + j * 1024, unroll=16)
            return carry

        lax.fori_loop(0, JCH, jbody, 0)

    def issue_stores(k, buf_t):
        cg = wid * PER_W + k
        l = cg // (B // CH)
        b0 = lax.rem(cg, B // CH) * CH
        for t in range(T):
            for i in range(2):
                dst = l * LSTRIDE + i * ISTRIDE + t * (32 * 1024) + b0 * 8
                pltpu.async_copy(
                    tr_v.at[buf_t, pl.ds(i * (TRCH // 2), TRCH // 2)],
                    out_hbm.at[pl.ds(dst, TRCH // 2)],
                    ssem[buf_t],
                )

    def drain_stores(buf_t):
        for _ in range(T):
            for i in range(2):
                pltpu.make_async_copy(
                    tr_v.at[buf_t, pl.ds(i * (TRCH // 2), TRCH // 2)],
                    out_hbm.at[pl.ds(0, TRCH // 2)],
                    ssem[buf_t],
                ).wait()

    def chunk_step(k, buf_r, buf_t, drain):
        fetch(k + 1, 1 - buf_r)
        if drain:
            drain_stores(buf_t)
        wait_gather(buf_r)
        transpose_chunk(buf_r, buf_t)
        issue_stores(k, buf_t)

    fetch(0, 0)
    for k in range(6):
        chunk_step(k, k & 1, k % 4, drain=k >= 4)

    def quad_body(kk, carry):
        for q in range(4):
            k = 4 * kk + 2 + q
            chunk_step(k, q & 1, (2 + q) % 4, drain=True)
        return carry

    lax.fori_loop(1, (PER_W - 6) // 4 + 1, quad_body, 0)

    for buf_t in range(4):
        drain_stores(buf_t)
    wait_gather(0)


def kernel(input, weight):
    w_lin = _tc_prep(jnp.transpose(weight)).reshape(VOCAB, D)
    idx_flat = input.T.reshape(N)
    out_flat = _sc_embed(w_lin, idx_flat)
    out5 = out_flat.reshape(L, D // 8, NJ, 8, 128)
    t1 = jnp.transpose(out5, (2, 4, 0, 1, 3))
    return t1.reshape(T * B, L, D)


# speedup vs baseline: 1.0188x; 1.0188x over previous
import functools

import jax
import jax.numpy as jnp
from jax import lax
from jax.experimental import pallas as pl
from jax.experimental.pallas import tpu as pltpu
from jax.experimental.pallas import tpu_sc as plsc

T = 4
B = 4096
L = 200
D = 16
N = B * L

NC = 2
NS = 16
NW = NC * NS
CH = 1024
NCHUNK_TOTAL = N // CH
PER_W = NCHUNK_TOTAL // NW
NBUF = 2
SCALE = 1.0 / T

_mesh = plsc.VectorSubcoreMesh(
    core_axis_name="c", subcore_axis_name="s", num_cores=NC, num_subcores=NS
)

VOCAB = 1000000
_TC_BC = 16384


def _tc_prep_body(wt_ref, out_ref):
    x = wt_ref[...] * jnp.float32(SCALE)
    xt = jnp.transpose(x)
    x3 = xt.reshape(_TC_BC // 8, 8, D)
    parts = [x3[:, k, :] for k in range(8)]
    out_ref[...] = jnp.concatenate(parts, axis=1)


def _tc_prep(wt):
    nb = pl.cdiv(VOCAB, _TC_BC)
    return pl.pallas_call(
        _tc_prep_body,
        grid=(nb,),
        in_specs=[pl.BlockSpec((D, _TC_BC), lambda i: (0, i))],
        out_specs=pl.BlockSpec((_TC_BC // 8, 128), lambda i: (i, 0)),
        out_shape=jax.ShapeDtypeStruct((VOCAB // 8, 128), jnp.float32),
    )(wt)


NJ = T * B // 128
JCH = CH // 128


TRCH = 2 * JCH * 8 * 128
LSTRIDE = (D // 8) * NJ * 8 * 128
ISTRIDE = NJ * 8 * 128


@functools.partial(
    pl.kernel,
    out_type=jax.ShapeDtypeStruct((L * D * T * B,), jnp.float32),
    mesh=_mesh,
    scratch_types=[
        pltpu.VMEM((NBUF, CH), jnp.int32),
        pltpu.VMEM((NBUF, CH, D), jnp.float32),
        pltpu.VMEM((2 * NBUF, TRCH), jnp.float32),
        pltpu.SemaphoreType.DMA,
        pltpu.SemaphoreType.DMA,
        pltpu.SemaphoreType.DMA,
        pltpu.SemaphoreType.DMA,
        pltpu.SemaphoreType.DMA,
        pltpu.SemaphoreType.DMA,
    ],
    compiler_params=pltpu.CompilerParams(
        use_tc_tiling_on_sc=False, needs_layout_passes=False
    ),
)
def _sc_embed(w_hbm, idx_hbm, out_hbm, idx_v, rows_v, tr_v, g0, g1, s0, s1, s2, s3):
    wid = lax.axis_index("s") * NC + lax.axis_index("c")
    gsem = (g0, g1)
    ssem = (s0, s1, s2, s3)
    lane_iota = lax.iota(jnp.int32, 16)
    base_vec = (
        lax.shift_right_logical(lane_iota, 3) * (JCH * 8 * 128)
        + lax.bitwise_and(lane_iota, 7) * 128
    )

    def fetch(k, buf):
        cg = wid * PER_W + lax.rem(k, PER_W)
        pltpu.sync_copy(idx_hbm.at[pl.ds(cg * CH, CH)], idx_v.at[buf])
        pltpu.async_copy(w_hbm.at[idx_v.at[buf]], rows_v.at[buf], gsem[buf])

    def wait_gather(buf):
        pltpu.make_async_copy(
            w_hbm.at[idx_v.at[buf]], rows_v.at[buf], gsem[buf]
        ).wait()

    def transpose_chunk(buf_r, buf_t):
        def jbody(j, carry):
            def tr_body(c, idx):
                row = rows_v[buf_r, j * 128 + c, :]
                plsc.store_scatter(tr_v.at[buf_t], [idx], row)
                return idx + 1

            lax.fori_loop(0, 128, tr_body, base_vec + j * 1024, unroll=16)
            return carry

        lax.fori_loop(0, JCH, jbody, 0)

    def issue_stores(k, buf_t):
        cg = wid * PER_W + k
        l = cg // (B // CH)
        b0 = lax.rem(cg, B // CH) * CH
        for t in range(T):
            for i in range(2):
                dst = l * LSTRIDE + i * ISTRIDE + t * (32 * 1024) + b0 * 8
                pltpu.async_copy(
                    tr_v.at[buf_t, pl.ds(i * (TRCH // 2), TRCH // 2)],
                    out_hbm.at[pl.ds(dst, TRCH // 2)],
                    ssem[buf_t],
                )

    def drain_stores(buf_t):
        for _ in range(T):
            for i in range(2):
                pltpu.make_async_copy(
                    tr_v.at[buf_t, pl.ds(i * (TRCH // 2), TRCH // 2)],
                    out_hbm.at[pl.ds(0, TRCH // 2)],
                    ssem[buf_t],
                ).wait()

    def chunk_step(k, buf_r, buf_t, drain):
        fetch(k + 1, 1 - buf_r)
        if drain:
            drain_stores(buf_t)
        wait_gather(buf_r)
        transpose_chunk(buf_r, buf_t)
        issue_stores(k, buf_t)

    fetch(0, 0)
    for k in range(5):
        chunk_step(k, k & 1, k % 4, drain=k >= 4)

    def quad_body(kk, carry):
        for q in range(4):
            k = 4 * kk + 1 + q
            chunk_step(k, (1 + q) & 1, (1 + q) % 4, drain=True)
        return carry

    lax.fori_loop(1, (PER_W - 5) // 4 + 1, quad_body, 0)

    for buf_t in range(4):
        drain_stores(buf_t)
    wait_gather(PER_W & 1)


def kernel(input, weight):
    w_lin = _tc_prep(jnp.transpose(weight)).reshape(VOCAB, D)
    idx_flat = input.T.reshape(N)
    out_flat = _sc_embed(w_lin, idx_flat)
    out5 = out_flat.reshape(L, D // 8, NJ, 8, 128)
    t1 = jnp.transpose(out5, (2, 4, 0, 1, 3))
    return t1.reshape(T * B, L, D)
